# baseline (speedup 1.0000x reference)
import functools

import jax
import jax.numpy as jnp
from jax import lax
from jax.experimental import pallas as pl
from jax.experimental.pallas import tpu as pltpu
from jax.experimental.pallas import tpu_sc as plsc

N = 10000
E = 320000
FEAT = 128
G = 128

NC = 2
NS = 16
NW = NC * NS
CHUNK = 64
CPW = 160
IBLK = 32
EPAD = NW * CPW * CHUNK
ACC_ROWS = 10240
RPS = ACC_ROWS // NS
PAD_DST = N


def _deg_body(edges_hbm, out_hbm, idx_v, ones_v, zbuf_v, acc_sh, dsem):
  c = lax.axis_index("c")
  s = lax.axis_index("s")
  w = c * NS + s

  pltpu.sync_copy(edges_hbm.at[1, pl.ds(w * CPW, CPW)], idx_v)

  def _fill(i, _):
    ones_v[pl.ds(i * 16, 16)] = jnp.ones((16,), jnp.float32)
    return 0

  lax.fori_loop(0, CHUNK // 16, _fill, 0)

  def _zero(i, _):
    zbuf_v[pl.ds(i * 16, 16)] = jnp.zeros((16,), jnp.float32)
    return 0

  lax.fori_loop(0, RPS // 16, _zero, 0)

  pltpu.sync_copy(zbuf_v, acc_sh.at[pl.ds(s * RPS, RPS)])
  plsc.subcore_barrier()

  def _grp(g, _):
    for j in range(16):
      pltpu.async_copy(ones_v, acc_sh.at[idx_v.at[g * 16 + j]], dsem,
                       add=True)
    for j in range(16):
      pltpu.make_async_copy(ones_v, acc_sh.at[idx_v.at[g * 16 + j]],
                            dsem).wait()
    return 0

  lax.fori_loop(0, CPW // 16, _grp, 0)
  plsc.subcore_barrier()

  pltpu.sync_copy(acc_sh.at[pl.ds(s * RPS, RPS)],
                  out_hbm.at[c, pl.ds(s * RPS, RPS)])


@jax.jit
def _deg_call(edges2d):
  return pl.kernel(
      _deg_body,
      out_type=jax.ShapeDtypeStruct((NC, ACC_ROWS), jnp.float32),
      mesh=plsc.VectorSubcoreMesh(core_axis_name="c", subcore_axis_name="s"),
      scratch_types=[
          pltpu.VMEM((CPW, CHUNK), jnp.int32),
          pltpu.VMEM((CHUNK,), jnp.float32),
          pltpu.VMEM((RPS,), jnp.float32),
          pltpu.VMEM_SHARED((ACC_ROWS,), jnp.float32),
          pltpu.SemaphoreType.DMA,
      ],
  )(edges2d)


def _conv_body(ht_hbm, edges_hbm, out_hbm,
               sidx_v, didx_v, rows0_v, rows1_v, rows2_v, acc_sh,
               gsem0, gsem1, gsem2, ssem0, ssem1, ssem2):
  c = lax.axis_index("c")
  s = lax.axis_index("s")
  w = c * NS + s

  def _zero(i, _):
    r = i // 8
    q = i % 8
    rows0_v[r, pl.ds(q * 16, 16)] = jnp.zeros((16,), jnp.float32)
    return 0

  lax.fori_loop(0, CHUNK * 8, _zero, 0)
  for k in range(RPS // CHUNK):
    pltpu.sync_copy(rows0_v, acc_sh.at[pl.ds(s * RPS + k * CHUNK, CHUNK)])
  plsc.subcore_barrier()

  bufs = (rows0_v, rows1_v, rows2_v)
  gsems = (gsem0, gsem1, gsem2)
  ssems = (ssem0, ssem1, ssem2)

  def _gather(j):
    k = j % 3
    pltpu.async_copy(ht_hbm.at[sidx_v.at[j]], bufs[k], gsems[k])

  def _gwait(j):
    k = j % 3
    pltpu.make_async_copy(ht_hbm.at[sidx_v.at[j]], bufs[k], gsems[k]).wait()

  def _scatter(j):
    k = j % 3
    pltpu.async_copy(bufs[k], acc_sh.at[didx_v.at[j]], ssems[k], add=True)

  def _swait(j):
    k = j % 3
    pltpu.make_async_copy(bufs[k], acc_sh.at[didx_v.at[j]], ssems[k]).wait()

  def _block(b, _):
    base = w * CPW + b * IBLK
    pltpu.sync_copy(edges_hbm.at[0, pl.ds(base, IBLK)], sidx_v)
    pltpu.sync_copy(edges_hbm.at[1, pl.ds(base, IBLK)], didx_v)
    for j in range(3):
      _gather(j)
    for j in range(IBLK):
      if j >= 2 and j + 1 < IBLK:
        _swait(j - 2)
        _gather(j + 1)
      _gwait(j)
      _scatter(j)
    for j in range(IBLK - 3, IBLK):
      _swait(j)
    return 0

  lax.fori_loop(0, CPW // IBLK, _block, 0)
  plsc.subcore_barrier()

  pltpu.sync_copy(acc_sh.at[pl.ds(s * RPS, RPS)],
                  out_hbm.at[c, pl.ds(s * RPS, RPS)])


@jax.jit
def _conv_call(ht, edges2d):
  return pl.kernel(
      _conv_body,
      out_type=jax.ShapeDtypeStruct((NC, ACC_ROWS, FEAT), jnp.float32),
      mesh=plsc.VectorSubcoreMesh(core_axis_name="c", subcore_axis_name="s"),
      scratch_types=[
          pltpu.VMEM((IBLK, CHUNK), jnp.int32),
          pltpu.VMEM((IBLK, CHUNK), jnp.int32),
          pltpu.VMEM((CHUNK, FEAT), jnp.float32),
          pltpu.VMEM((CHUNK, FEAT), jnp.float32),
          pltpu.VMEM((CHUNK, FEAT), jnp.float32),
          pltpu.VMEM_SHARED((ACC_ROWS, FEAT), jnp.float32),
          pltpu.SemaphoreType.DMA,
          pltpu.SemaphoreType.DMA,
          pltpu.SemaphoreType.DMA,
          pltpu.SemaphoreType.DMA,
          pltpu.SemaphoreType.DMA,
          pltpu.SemaphoreType.DMA,
      ],
  )(ht, edges2d)


def _tc1a_body(x_ref, w1_ref, h_ref):
  h_ref[...] = jnp.dot(x_ref[...], w1_ref[...],
                       preferred_element_type=jnp.float32)


@jax.jit
def _tc1a_call(x, W1):
  return pl.pallas_call(
      _tc1a_body,
      out_shape=jax.ShapeDtypeStruct((N, FEAT), jnp.float32),
  )(x, W1)


def _tc1b_body(degp_ref, h_ref, dinv_ref, ht_ref):
  dv = degp_ref[...]
  deg = (dv[0] + dv[1] + 1.0)[:N]
  dinv = lax.rsqrt(deg)[:, None]
  dinv_ref[...] = dinv
  ht_ref[...] = h_ref[...] * dinv


@jax.jit
def _tc1b_call(degp, h1):
  return pl.pallas_call(
      _tc1b_body,
      out_shape=[
          jax.ShapeDtypeStruct((N, 1), jnp.float32),
          jax.ShapeDtypeStruct((N, FEAT), jnp.float32),
      ],
  )(degp, h1)


def _bn_relu(conv, gamma, beta):
  mu = jnp.mean(conv, axis=0, keepdims=True)
  xc = conv - mu
  var = jnp.mean(xc * xc, axis=0, keepdims=True)
  return jnp.maximum(xc * lax.rsqrt(var + 1e-5) * gamma + beta, 0.0)


def _tc2_body(agg_ref, ht_ref, dinv_ref, g_ref, b_ref, w_ref, out_ref):
  dinv = dinv_ref[...]
  conv = (agg_ref[0, :N] + agg_ref[1, :N] + ht_ref[...]) * dinv
  y = _bn_relu(conv, g_ref[...], b_ref[...])
  out_ref[...] = jnp.dot(y, w_ref[...],
                         preferred_element_type=jnp.float32) * dinv


@jax.jit
def _tc2_call(agg, ht, dinv, gamma, beta, Wn):
  return pl.pallas_call(
      _tc2_body,
      out_shape=jax.ShapeDtypeStruct((N, FEAT), jnp.float32),
  )(agg, ht, dinv, gamma, beta, Wn)


def _tc3_body(agg_ref, ht_ref, dinv_ref, g_ref, b_ref, batch_ref,
              f1w_ref, f1b_ref, f2w_ref, f2b_ref, out_ref):
  conv = (agg_ref[0, :N] + agg_ref[1, :N] + ht_ref[...]) * dinv_ref[...]
  y = _bn_relu(conv, g_ref[...], b_ref[...])
  gid = lax.broadcasted_iota(jnp.int32, (1, G), 1)
  m = (batch_ref[...] == gid).astype(jnp.float32)
  dn = (((0,), (0,)), ((), ()))
  sums = lax.dot_general(m, y, dn, preferred_element_type=jnp.float32)
  ones = jnp.ones((N, 1), jnp.float32)
  counts = lax.dot_general(m, ones, dn, preferred_element_type=jnp.float32)
  pooled = sums / jnp.maximum(counts, 1.0)
  a = jnp.maximum(
      jnp.dot(pooled, f1w_ref[...], preferred_element_type=jnp.float32)
      + f1b_ref[...], 0.0)
  out_ref[...] = jnp.dot(
      a, f2w_ref[...], preferred_element_type=jnp.float32) + f2b_ref[...]


@jax.jit
def _tc3_call(agg, ht, dinv, gamma, beta, batch2d,
              fc1Wp, fc1bp, fc2Wp, fc2bp):
  return pl.pallas_call(
      _tc3_body,
      out_shape=jax.ShapeDtypeStruct((G, FEAT), jnp.float32),
  )(agg, ht, dinv, gamma, beta, batch2d, fc1Wp, fc1bp, fc2Wp, fc2bp)


def kernel(x, edge_index, batch, W1, b1, gamma1, beta1, W2, b2, gamma2,
           beta2, fc1_W, fc1_b, fc2_W, fc2_b):
  pad = EPAD - E
  ar = jnp.arange(pad, dtype=jnp.int32)
  pad2 = jnp.stack([ar % N, PAD_DST + (ar % (ACC_ROWS - N))])
  edges2d = jnp.concatenate([edge_index, pad2], axis=1).reshape(
      2, NW * CPW, CHUNK)

  degp = _deg_call(edges2d)
  h1 = _tc1a_call(x, W1)
  dinv, ht1 = _tc1b_call(degp, h1)

  agg1 = _conv_call(ht1, edges2d)
  ht2 = _tc2_call(agg1, ht1, dinv, gamma1[None, :], beta1[None, :], W2)

  agg2 = _conv_call(ht2, edges2d)

  fc1Wp = jnp.pad(fc1_W, ((0, 0), (0, FEAT - fc1_W.shape[1])))
  fc1bp = jnp.pad(fc1_b, (0, FEAT - fc1_b.shape[0]))[None, :]
  fc2Wp = jnp.pad(fc2_W, ((0, FEAT - fc2_W.shape[0]),
                          (0, FEAT - fc2_W.shape[1])))
  fc2bp = jnp.pad(fc2_b, (0, FEAT - fc2_b.shape[0]))[None, :]
  outp = _tc3_call(agg2, ht2, dinv,
                   gamma2[None, :], beta2[None, :], batch[:, None],
                   fc1Wp, fc1bp, fc2Wp, fc2bp)
  return outp[:, :fc2_W.shape[1]]


# speedup vs baseline: 1.0558x; 1.0558x over previous
import functools

import jax
import jax.numpy as jnp
from jax import lax
from jax.experimental import pallas as pl
from jax.experimental.pallas import tpu as pltpu
from jax.experimental.pallas import tpu_sc as plsc

N = 10000
E = 320000
FEAT = 128
G = 128

NC = 2
NS = 16
NW = NC * NS
CHUNK = 64
CPW = 160
IBLK = 32
EPAD = NW * CPW * CHUNK
ACC_ROWS = 10240
RPS = ACC_ROWS // NS
PAD_DST = N


def _deg_body(edges_hbm, out_hbm, idx_v, ones_v, zbuf_v, acc_sh, dsem):
  c = lax.axis_index("c")
  s = lax.axis_index("s")
  w = c * NS + s

  pltpu.sync_copy(edges_hbm.at[1, pl.ds(w * CPW, CPW)], idx_v)

  def _fill(i, _):
    ones_v[pl.ds(i * 16, 16)] = jnp.ones((16,), jnp.float32)
    return 0

  lax.fori_loop(0, CHUNK // 16, _fill, 0)

  def _zero(i, _):
    zbuf_v[pl.ds(i * 16, 16)] = jnp.zeros((16,), jnp.float32)
    return 0

  lax.fori_loop(0, RPS // 16, _zero, 0)

  pltpu.sync_copy(zbuf_v, acc_sh.at[pl.ds(s * RPS, RPS)])
  plsc.subcore_barrier()

  def _grp(g, _):
    for j in range(16):
      pltpu.async_copy(ones_v, acc_sh.at[idx_v.at[g * 16 + j]], dsem,
                       add=True)
    for j in range(16):
      pltpu.make_async_copy(ones_v, acc_sh.at[idx_v.at[g * 16 + j]],
                            dsem).wait()
    return 0

  lax.fori_loop(0, CPW // 16, _grp, 0)
  plsc.subcore_barrier()

  pltpu.sync_copy(acc_sh.at[pl.ds(s * RPS, RPS)],
                  out_hbm.at[c, pl.ds(s * RPS, RPS)])


@jax.jit
def _deg_call(edges2d):
  return pl.kernel(
      _deg_body,
      out_type=jax.ShapeDtypeStruct((NC, ACC_ROWS), jnp.float32),
      mesh=plsc.VectorSubcoreMesh(core_axis_name="c", subcore_axis_name="s"),
      scratch_types=[
          pltpu.VMEM((CPW, CHUNK), jnp.int32),
          pltpu.VMEM((CHUNK,), jnp.float32),
          pltpu.VMEM((RPS,), jnp.float32),
          pltpu.VMEM_SHARED((ACC_ROWS,), jnp.float32),
          pltpu.SemaphoreType.DMA,
      ],
  )(edges2d)


def _conv_body(ht_hbm, edges_hbm, out_hbm,
               sidx_v, didx_v, rows0_v, rows1_v, rows2_v, acc_sh,
               gsem0, gsem1, gsem2):
  c = lax.axis_index("c")
  s = lax.axis_index("s")
  w = c * NS + s

  def _zero(i, _):
    r = i // 8
    q = i % 8
    rows0_v[r, pl.ds(q * 16, 16)] = jnp.zeros((16,), jnp.float32)
    return 0

  lax.fori_loop(0, CHUNK * 8, _zero, 0)
  for k in range(RPS // CHUNK):
    pltpu.sync_copy(rows0_v, acc_sh.at[pl.ds(s * RPS + k * CHUNK, CHUNK)])
  plsc.subcore_barrier()

  bufs = (rows0_v, rows1_v, rows2_v)
  sems = (gsem0, gsem1, gsem2)

  def _block(b, _):
    base = w * CPW + b * IBLK
    pltpu.sync_copy(edges_hbm.at[0, pl.ds(base, IBLK)], sidx_v)
    pltpu.sync_copy(edges_hbm.at[1, pl.ds(base, IBLK)], didx_v)
    pltpu.async_copy(ht_hbm.at[sidx_v.at[0]], bufs[0], sems[0])
    pltpu.async_copy(ht_hbm.at[sidx_v.at[1]], bufs[1], sems[1])
    for j in range(IBLK):
      if j + 2 < IBLK:
        k = (j + 2) % 3
        pltpu.async_copy(ht_hbm.at[sidx_v.at[j + 2]], bufs[k], sems[k])
      m = j % 3
      pltpu.make_async_copy(ht_hbm.at[sidx_v.at[j]], bufs[m], sems[m]).wait()
      pltpu.sync_copy(bufs[m], acc_sh.at[didx_v.at[j]], add=True)
    return 0

  lax.fori_loop(0, CPW // IBLK, _block, 0)
  plsc.subcore_barrier()

  pltpu.sync_copy(acc_sh.at[pl.ds(s * RPS, RPS)],
                  out_hbm.at[c, pl.ds(s * RPS, RPS)])


@jax.jit
def _conv_call(ht, edges2d):
  return pl.kernel(
      _conv_body,
      out_type=jax.ShapeDtypeStruct((NC, ACC_ROWS, FEAT), jnp.float32),
      mesh=plsc.VectorSubcoreMesh(core_axis_name="c", subcore_axis_name="s"),
      scratch_types=[
          pltpu.VMEM((IBLK, CHUNK), jnp.int32),
          pltpu.VMEM((IBLK, CHUNK), jnp.int32),
          pltpu.VMEM((CHUNK, FEAT), jnp.float32),
          pltpu.VMEM((CHUNK, FEAT), jnp.float32),
          pltpu.VMEM((CHUNK, FEAT), jnp.float32),
          pltpu.VMEM_SHARED((ACC_ROWS, FEAT), jnp.float32),
          pltpu.SemaphoreType.DMA,
          pltpu.SemaphoreType.DMA,
          pltpu.SemaphoreType.DMA,
      ],
  )(ht, edges2d)


def _tc1a_body(x_ref, w1_ref, h_ref):
  h_ref[...] = jnp.dot(x_ref[...], w1_ref[...],
                       preferred_element_type=jnp.float32)


@jax.jit
def _tc1a_call(x, W1):
  return pl.pallas_call(
      _tc1a_body,
      out_shape=jax.ShapeDtypeStruct((N, FEAT), jnp.float32),
  )(x, W1)


def _tc1b_body(degp_ref, h_ref, dinv_ref, ht_ref):
  dv = degp_ref[...]
  deg = (dv[0] + dv[1] + 1.0)[:N]
  dinv = lax.rsqrt(deg)[:, None]
  dinv_ref[...] = dinv
  ht_ref[...] = h_ref[...] * dinv


@jax.jit
def _tc1b_call(degp, h1):
  return pl.pallas_call(
      _tc1b_body,
      out_shape=[
          jax.ShapeDtypeStruct((N, 1), jnp.float32),
          jax.ShapeDtypeStruct((N, FEAT), jnp.float32),
      ],
  )(degp, h1)


def _bn_relu(conv, gamma, beta):
  mu = jnp.mean(conv, axis=0, keepdims=True)
  xc = conv - mu
  var = jnp.mean(xc * xc, axis=0, keepdims=True)
  return jnp.maximum(xc * lax.rsqrt(var + 1e-5) * gamma + beta, 0.0)


def _tc2_body(agg_ref, ht_ref, dinv_ref, g_ref, b_ref, w_ref, out_ref):
  dinv = dinv_ref[...]
  conv = (agg_ref[0, :N] + agg_ref[1, :N] + ht_ref[...]) * dinv
  y = _bn_relu(conv, g_ref[...], b_ref[...])
  out_ref[...] = jnp.dot(y, w_ref[...],
                         preferred_element_type=jnp.float32) * dinv


@jax.jit
def _tc2_call(agg, ht, dinv, gamma, beta, Wn):
  return pl.pallas_call(
      _tc2_body,
      out_shape=jax.ShapeDtypeStruct((N, FEAT), jnp.float32),
  )(agg, ht, dinv, gamma, beta, Wn)


def _tc3_body(agg_ref, ht_ref, dinv_ref, g_ref, b_ref, batch_ref,
              f1w_ref, f1b_ref, f2w_ref, f2b_ref, out_ref):
  conv = (agg_ref[0, :N] + agg_ref[1, :N] + ht_ref[...]) * dinv_ref[...]
  y = _bn_relu(conv, g_ref[...], b_ref[...])
  gid = lax.broadcasted_iota(jnp.int32, (1, G), 1)
  m = (batch_ref[...] == gid).astype(jnp.float32)
  dn = (((0,), (0,)), ((), ()))
  sums = lax.dot_general(m, y, dn, preferred_element_type=jnp.float32)
  ones = jnp.ones((N, 1), jnp.float32)
  counts = lax.dot_general(m, ones, dn, preferred_element_type=jnp.float32)
  pooled = sums / jnp.maximum(counts, 1.0)
  a = jnp.maximum(
      jnp.dot(pooled, f1w_ref[...], preferred_element_type=jnp.float32)
      + f1b_ref[...], 0.0)
  out_ref[...] = jnp.dot(
      a, f2w_ref[...], preferred_element_type=jnp.float32) + f2b_ref[...]


@jax.jit
def _tc3_call(agg, ht, dinv, gamma, beta, batch2d,
              fc1Wp, fc1bp, fc2Wp, fc2bp):
  return pl.pallas_call(
      _tc3_body,
      out_shape=jax.ShapeDtypeStruct((G, FEAT), jnp.float32),
  )(agg, ht, dinv, gamma, beta, batch2d, fc1Wp, fc1bp, fc2Wp, fc2bp)


def kernel(x, edge_index, batch, W1, b1, gamma1, beta1, W2, b2, gamma2,
           beta2, fc1_W, fc1_b, fc2_W, fc2_b):
  pad = EPAD - E
  ar = jnp.arange(pad, dtype=jnp.int32)
  pad2 = jnp.stack([ar % N, PAD_DST + (ar % (ACC_ROWS - N))])
  edges2d = jnp.concatenate([edge_index, pad2], axis=1).reshape(
      2, NW * CPW, CHUNK)

  degp = _deg_call(edges2d)
  h1 = _tc1a_call(x, W1)
  dinv, ht1 = _tc1b_call(degp, h1)

  agg1 = _conv_call(ht1, edges2d)
  ht2 = _tc2_call(agg1, ht1, dinv, gamma1[None, :], beta1[None, :], W2)

  agg2 = _conv_call(ht2, edges2d)

  fc1Wp = jnp.pad(fc1_W, ((0, 0), (0, FEAT - fc1_W.shape[1])))
  fc1bp = jnp.pad(fc1_b, (0, FEAT - fc1_b.shape[0]))[None, :]
  fc2Wp = jnp.pad(fc2_W, ((0, FEAT - fc2_W.shape[0]),
                          (0, FEAT - fc2_W.shape[1])))
  fc2bp = jnp.pad(fc2_b, (0, FEAT - fc2_b.shape[0]))[None, :]
  outp = _tc3_call(agg2, ht2, dinv,
                   gamma2[None, :], beta2[None, :], batch[:, None],
                   fc1Wp, fc1bp, fc2Wp, fc2bp)
  return outp[:, :fc2_W.shape[1]]


# speedup vs baseline: 1.0928x; 1.0350x over previous
import functools

import jax
import jax.numpy as jnp
import numpy as np
from jax import lax
from jax.experimental import pallas as pl
from jax.experimental.pallas import tpu as pltpu
from jax.experimental.pallas import tpu_sc as plsc

N = 10000
E = 320000
FEAT = 128
G = 128

NC = 2
NS = 16
NW = NC * NS
CHUNK = 64
CPW = 160
IBLK = 32
EPAD = NW * CPW * CHUNK
ACC_ROWS = 10240
RPS = ACC_ROWS // NS
PAD_DST = N
NBLK = CPW // IBLK

_AR = np.arange(EPAD - E, dtype=np.int32)
_PAD2 = np.stack([_AR % N, PAD_DST + (_AR % (ACC_ROWS - N))])


def _deg_body(edges_hbm, out_hbm, idx_v, ones_v, zbuf_v, acc_sh, dsem):
  c = lax.axis_index("c")
  s = lax.axis_index("s")
  w = c * NS + s

  pltpu.sync_copy(edges_hbm.at[1, pl.ds(w * CPW, CPW)], idx_v)

  def _fill(i, _):
    ones_v[pl.ds(i * 16, 16)] = jnp.ones((16,), jnp.float32)
    return 0

  lax.fori_loop(0, CHUNK // 16, _fill, 0)

  def _zero(i, _):
    zbuf_v[pl.ds(i * 16, 16)] = jnp.zeros((16,), jnp.float32)
    return 0

  lax.fori_loop(0, RPS // 16, _zero, 0)

  pltpu.sync_copy(zbuf_v, acc_sh.at[pl.ds(s * RPS, RPS)])
  plsc.subcore_barrier()

  def _grp(g, _):
    for j in range(16):
      pltpu.async_copy(ones_v, acc_sh.at[idx_v.at[g * 16 + j]], dsem,
                       add=True)
    for j in range(16):
      pltpu.make_async_copy(ones_v, acc_sh.at[idx_v.at[g * 16 + j]],
                            dsem).wait()
    return 0

  lax.fori_loop(0, CPW // 16, _grp, 0)
  plsc.subcore_barrier()

  pltpu.sync_copy(acc_sh.at[pl.ds(s * RPS, RPS)],
                  out_hbm.at[c, pl.ds(s * RPS, RPS)])


@jax.jit
def _deg_call(edges2d):
  return pl.kernel(
      _deg_body,
      out_type=jax.ShapeDtypeStruct((NC, ACC_ROWS), jnp.float32),
      mesh=plsc.VectorSubcoreMesh(core_axis_name="c", subcore_axis_name="s"),
      scratch_types=[
          pltpu.VMEM((CPW, CHUNK), jnp.int32),
          pltpu.VMEM((CHUNK,), jnp.float32),
          pltpu.VMEM((RPS,), jnp.float32),
          pltpu.VMEM_SHARED((ACC_ROWS,), jnp.float32),
          pltpu.SemaphoreType.DMA,
      ],
  )(edges2d)


def _conv_body(ht_hbm, edges_hbm, out_hbm,
               sidxA_v, didxA_v, sidxB_v, didxB_v,
               rows0_v, rows1_v, rows2_v, acc_sh,
               gsem0, gsem1, gsem2, isemA, isemB):
  c = lax.axis_index("c")
  s = lax.axis_index("s")
  w = c * NS + s

  def _zero(i, _):
    r = i // 8
    q = i % 8
    rows0_v[r, pl.ds(q * 16, 16)] = jnp.zeros((16,), jnp.float32)
    return 0

  lax.fori_loop(0, CHUNK * 8, _zero, 0)
  for k in range(RPS // CHUNK):
    pltpu.sync_copy(rows0_v, acc_sh.at[pl.ds(s * RPS + k * CHUNK, CHUNK)])
  plsc.subcore_barrier()

  bufs = (rows0_v, rows1_v, rows2_v)
  sems = (gsem0, gsem1, gsem2)
  idxs = ((sidxA_v, didxA_v, isemA), (sidxB_v, didxB_v, isemB))

  def _stage(b, sync):
    si, di, sem = idxs[b % 2]
    base = w * CPW + b * IBLK
    if sync:
      pltpu.sync_copy(edges_hbm.at[0, pl.ds(base, IBLK)], si)
      pltpu.sync_copy(edges_hbm.at[1, pl.ds(base, IBLK)], di)
    else:
      pltpu.async_copy(edges_hbm.at[0, pl.ds(base, IBLK)], si, sem)
      pltpu.async_copy(edges_hbm.at[1, pl.ds(base, IBLK)], di, sem)

  def _stage_wait(b):
    si, di, sem = idxs[b % 2]
    base = w * CPW + b * IBLK
    pltpu.make_async_copy(edges_hbm.at[0, pl.ds(base, IBLK)], si, sem).wait()
    pltpu.make_async_copy(edges_hbm.at[1, pl.ds(base, IBLK)], di, sem).wait()

  _stage(0, sync=True)
  for b in range(NBLK):
    sidx_v, didx_v, _ = idxs[b % 2]
    if b + 1 < NBLK:
      _stage(b + 1, sync=False)
    if b > 0:
      pass
    pltpu.async_copy(ht_hbm.at[sidx_v.at[0]], bufs[0], sems[0])
    pltpu.async_copy(ht_hbm.at[sidx_v.at[1]], bufs[1], sems[1])
    for j in range(IBLK):
      if j + 2 < IBLK:
        k = (j + 2) % 3
        pltpu.async_copy(ht_hbm.at[sidx_v.at[j + 2]], bufs[k], sems[k])
      m = j % 3
      pltpu.make_async_copy(ht_hbm.at[sidx_v.at[j]], bufs[m], sems[m]).wait()
      pltpu.sync_copy(bufs[m], acc_sh.at[didx_v.at[j]], add=True)
    if b + 1 < NBLK:
      _stage_wait(b + 1)
  plsc.subcore_barrier()

  pltpu.sync_copy(acc_sh.at[pl.ds(s * RPS, RPS)],
                  out_hbm.at[c, pl.ds(s * RPS, RPS)])


@jax.jit
def _conv_call(ht, edges2d):
  return pl.kernel(
      _conv_body,
      out_type=jax.ShapeDtypeStruct((NC, ACC_ROWS, FEAT), jnp.float32),
      mesh=plsc.VectorSubcoreMesh(core_axis_name="c", subcore_axis_name="s"),
      scratch_types=[
          pltpu.VMEM((IBLK, CHUNK), jnp.int32),
          pltpu.VMEM((IBLK, CHUNK), jnp.int32),
          pltpu.VMEM((IBLK, CHUNK), jnp.int32),
          pltpu.VMEM((IBLK, CHUNK), jnp.int32),
          pltpu.VMEM((CHUNK, FEAT), jnp.float32),
          pltpu.VMEM((CHUNK, FEAT), jnp.float32),
          pltpu.VMEM((CHUNK, FEAT), jnp.float32),
          pltpu.VMEM_SHARED((ACC_ROWS, FEAT), jnp.float32),
          pltpu.SemaphoreType.DMA,
          pltpu.SemaphoreType.DMA,
          pltpu.SemaphoreType.DMA,
          pltpu.SemaphoreType.DMA,
          pltpu.SemaphoreType.DMA,
      ],
  )(ht, edges2d)


def _tc1a_body(x_ref, w1_ref, h_ref):
  h_ref[...] = jnp.dot(x_ref[...], w1_ref[...],
                       preferred_element_type=jnp.float32)


@jax.jit
def _tc1a_call(x, W1):
  return pl.pallas_call(
      _tc1a_body,
      out_shape=jax.ShapeDtypeStruct((N, FEAT), jnp.float32),
  )(x, W1)


def _tc1b_body(degp_ref, h_ref, dinv_ref, ht_ref):
  dv = degp_ref[...]
  deg = (dv[0] + dv[1] + 1.0)[:N]
  dinv = lax.rsqrt(deg)[:, None]
  dinv_ref[...] = dinv
  ht_ref[...] = h_ref[...] * dinv


@jax.jit
def _tc1b_call(degp, h1):
  return pl.pallas_call(
      _tc1b_body,
      out_shape=[
          jax.ShapeDtypeStruct((N, 1), jnp.float32),
          jax.ShapeDtypeStruct((N, FEAT), jnp.float32),
      ],
  )(degp, h1)


def _bn_relu(conv, gamma, beta):
  mu = jnp.mean(conv, axis=0, keepdims=True)
  xc = conv - mu
  var = jnp.mean(xc * xc, axis=0, keepdims=True)
  return jnp.maximum(xc * lax.rsqrt(var + 1e-5) * gamma + beta, 0.0)


def _tc2_body(agg_ref, ht_ref, dinv_ref, g_ref, b_ref, w_ref, out_ref):
  dinv = dinv_ref[...]
  conv = (agg_ref[0, :N] + agg_ref[1, :N] + ht_ref[...]) * dinv
  y = _bn_relu(conv, g_ref[...], b_ref[...])
  out_ref[...] = jnp.dot(y, w_ref[...],
                         preferred_element_type=jnp.float32) * dinv


@jax.jit
def _tc2_call(agg, ht, dinv, gamma, beta, Wn):
  return pl.pallas_call(
      _tc2_body,
      out_shape=jax.ShapeDtypeStruct((N, FEAT), jnp.float32),
  )(agg, ht, dinv, gamma, beta, Wn)


def _tc3_body(agg_ref, ht_ref, dinv_ref, g_ref, b_ref, batch_ref,
              f1w_ref, f1b_ref, f2w_ref, f2b_ref, out_ref):
  conv = (agg_ref[0, :N] + agg_ref[1, :N] + ht_ref[...]) * dinv_ref[...]
  y = _bn_relu(conv, g_ref[...], b_ref[...])
  gid = lax.broadcasted_iota(jnp.int32, (1, G), 1)
  m = (batch_ref[...] == gid).astype(jnp.float32)
  dn = (((0,), (0,)), ((), ()))
  sums = lax.dot_general(m, y, dn, preferred_element_type=jnp.float32)
  ones = jnp.ones((N, 1), jnp.float32)
  counts = lax.dot_general(m, ones, dn, preferred_element_type=jnp.float32)
  pooled = sums / jnp.maximum(counts, 1.0)
  a = jnp.maximum(
      jnp.dot(pooled, f1w_ref[...], preferred_element_type=jnp.float32)
      + f1b_ref[...], 0.0)
  out_ref[...] = jnp.dot(
      a, f2w_ref[...], preferred_element_type=jnp.float32) + f2b_ref[...]


@jax.jit
def _tc3_call(agg, ht, dinv, gamma, beta, batch2d,
              fc1Wp, fc1bp, fc2Wp, fc2bp):
  return pl.pallas_call(
      _tc3_body,
      out_shape=jax.ShapeDtypeStruct((G, FEAT), jnp.float32),
  )(agg, ht, dinv, gamma, beta, batch2d, fc1Wp, fc1bp, fc2Wp, fc2bp)


def kernel(x, edge_index, batch, W1, b1, gamma1, beta1, W2, b2, gamma2,
           beta2, fc1_W, fc1_b, fc2_W, fc2_b):
  edges2d = jnp.concatenate(
      [edge_index, jnp.asarray(_PAD2)], axis=1).reshape(2, NW * CPW, CHUNK)

  degp = _deg_call(edges2d)
  h1 = _tc1a_call(x, W1)
  dinv, ht1 = _tc1b_call(degp, h1)

  agg1 = _conv_call(ht1, edges2d)
  ht2 = _tc2_call(agg1, ht1, dinv, gamma1[None, :], beta1[None, :], W2)

  agg2 = _conv_call(ht2, edges2d)

  fc1Wp = jnp.pad(fc1_W, ((0, 0), (0, FEAT - fc1_W.shape[1])))
  fc1bp = jnp.pad(fc1_b, (0, FEAT - fc1_b.shape[0]))[None, :]
  fc2Wp = jnp.pad(fc2_W, ((0, FEAT - fc2_W.shape[0]),
                          (0, FEAT - fc2_W.shape[1])))
  fc2bp = jnp.pad(fc2_b, (0, FEAT - fc2_b.shape[0]))[None, :]
  outp = _tc3_call(agg2, ht2, dinv,
                   gamma2[None, :], beta2[None, :], batch[:, None],
                   fc1Wp, fc1bp, fc2Wp, fc2bp)
  return outp[:, :fc2_W.shape[1]]


# speedup vs baseline: 1.1331x; 1.0369x over previous
import functools

import jax
import jax.numpy as jnp
import numpy as np
from jax import lax
from jax.experimental import pallas as pl
from jax.experimental.pallas import tpu as pltpu
from jax.experimental.pallas import tpu_sc as plsc

N = 10000
E = 320000
FEAT = 128
G = 128

NC = 2
NS = 16
NW = NC * NS
CHUNK = 64
CPW = 160
IBLK = 32
EPAD = NW * CPW * CHUNK
ACC_ROWS = 10240
RPS = ACC_ROWS // NS
PAD_DST = N
NBLK = CPW // IBLK

_AR = np.arange(EPAD - E, dtype=np.int32)
_PAD2 = np.stack([_AR % N, PAD_DST + (_AR % (ACC_ROWS - N))])


def _deg_body(edges_hbm, out_hbm, idx_v, ones_v, zbuf_v, acc_sh, dsem):
  c = lax.axis_index("c")
  s = lax.axis_index("s")
  w = c * NS + s

  pltpu.sync_copy(edges_hbm.at[1, pl.ds(w * CPW, CPW)], idx_v)

  def _fill(i, _):
    ones_v[pl.ds(i * 16, 16)] = jnp.ones((16,), jnp.float32)
    return 0

  lax.fori_loop(0, CHUNK // 16, _fill, 0)

  def _zero(i, _):
    zbuf_v[pl.ds(i * 16, 16)] = jnp.zeros((16,), jnp.float32)
    return 0

  lax.fori_loop(0, RPS // 16, _zero, 0)

  pltpu.sync_copy(zbuf_v, acc_sh.at[pl.ds(s * RPS, RPS)])
  plsc.subcore_barrier()

  def _grp(g, _):
    for j in range(16):
      pltpu.async_copy(ones_v, acc_sh.at[idx_v.at[g * 16 + j]], dsem,
                       add=True)
    for j in range(16):
      pltpu.make_async_copy(ones_v, acc_sh.at[idx_v.at[g * 16 + j]],
                            dsem).wait()
    return 0

  lax.fori_loop(0, CPW // 16, _grp, 0)
  plsc.subcore_barrier()

  pltpu.sync_copy(acc_sh.at[pl.ds(s * RPS, RPS)],
                  out_hbm.at[c, pl.ds(s * RPS, RPS)])


@jax.jit
def _deg_call(edges2d):
  return pl.kernel(
      _deg_body,
      out_type=jax.ShapeDtypeStruct((NC, ACC_ROWS), jnp.float32),
      mesh=plsc.VectorSubcoreMesh(core_axis_name="c", subcore_axis_name="s"),
      scratch_types=[
          pltpu.VMEM((CPW, CHUNK), jnp.int32),
          pltpu.VMEM((CHUNK,), jnp.float32),
          pltpu.VMEM((RPS,), jnp.float32),
          pltpu.VMEM_SHARED((ACC_ROWS,), jnp.float32),
          pltpu.SemaphoreType.DMA,
      ],
  )(edges2d)


def _conv_body(ht_hbm, edges_hbm, out_hbm,
               sidxA_v, didxA_v, sidxB_v, didxB_v,
               rows0_v, rows1_v, rows2_v, acc_sh,
               gsem0, gsem1, gsem2, isemA, isemB):
  c = lax.axis_index("c")
  s = lax.axis_index("s")
  w = c * NS + s

  def _zero(i, _):
    r = i // 8
    q = i % 8
    rows0_v[r, pl.ds(q * 16, 16)] = jnp.zeros((16,), jnp.float32)
    return 0

  lax.fori_loop(0, CHUNK * 8, _zero, 0)
  for k in range(RPS // CHUNK):
    pltpu.sync_copy(rows0_v, acc_sh.at[pl.ds(s * RPS + k * CHUNK, CHUNK)])
  plsc.subcore_barrier()

  bufs = (rows0_v, rows1_v, rows2_v)
  sems = (gsem0, gsem1, gsem2)
  idxs = ((sidxA_v, didxA_v, isemA), (sidxB_v, didxB_v, isemB))

  def _stage(b, sync):
    si, di, sem = idxs[b % 2]
    base = w * CPW + b * IBLK
    if sync:
      pltpu.sync_copy(edges_hbm.at[0, pl.ds(base, IBLK)], si)
      pltpu.sync_copy(edges_hbm.at[1, pl.ds(base, IBLK)], di)
    else:
      pltpu.async_copy(edges_hbm.at[0, pl.ds(base, IBLK)], si, sem)
      pltpu.async_copy(edges_hbm.at[1, pl.ds(base, IBLK)], di, sem)

  def _stage_wait(b):
    si, di, sem = idxs[b % 2]
    base = w * CPW + b * IBLK
    pltpu.make_async_copy(edges_hbm.at[0, pl.ds(base, IBLK)], si, sem).wait()
    pltpu.make_async_copy(edges_hbm.at[1, pl.ds(base, IBLK)], di, sem).wait()

  def _gather(g):
    si = idxs[(g // IBLK) % 2][0]
    k = g % 3
    pltpu.async_copy(ht_hbm.at[si.at[g % IBLK]], bufs[k], sems[k])

  def _gwait(g):
    si = idxs[(g // IBLK) % 2][0]
    k = g % 3
    pltpu.make_async_copy(ht_hbm.at[si.at[g % IBLK]], bufs[k],
                          sems[k]).wait()

  _stage(0, sync=True)
  if NBLK > 1:
    _stage(1, sync=False)
  _gather(0)
  _gather(1)
  for jj in range(CPW):
    g2 = jj + 2
    if g2 < CPW:
      if g2 % IBLK == 0:
        _stage_wait(g2 // IBLK)
      _gather(g2)
    _gwait(jj)
    di = idxs[(jj // IBLK) % 2][1]
    pltpu.sync_copy(bufs[jj % 3], acc_sh.at[di.at[jj % IBLK]], add=True)
    if jj % IBLK == IBLK - 1 and jj // IBLK + 2 < NBLK:
      _stage(jj // IBLK + 2, sync=False)
  plsc.subcore_barrier()

  pltpu.sync_copy(acc_sh.at[pl.ds(s * RPS, RPS)],
                  out_hbm.at[c, pl.ds(s * RPS, RPS)])


@jax.jit
def _conv_call(ht, edges2d):
  return pl.kernel(
      _conv_body,
      out_type=jax.ShapeDtypeStruct((NC, ACC_ROWS, FEAT), jnp.float32),
      mesh=plsc.VectorSubcoreMesh(core_axis_name="c", subcore_axis_name="s"),
      scratch_types=[
          pltpu.VMEM((IBLK, CHUNK), jnp.int32),
          pltpu.VMEM((IBLK, CHUNK), jnp.int32),
          pltpu.VMEM((IBLK, CHUNK), jnp.int32),
          pltpu.VMEM((IBLK, CHUNK), jnp.int32),
          pltpu.VMEM((CHUNK, FEAT), jnp.float32),
          pltpu.VMEM((CHUNK, FEAT), jnp.float32),
          pltpu.VMEM((CHUNK, FEAT), jnp.float32),
          pltpu.VMEM_SHARED((ACC_ROWS, FEAT), jnp.float32),
          pltpu.SemaphoreType.DMA,
          pltpu.SemaphoreType.DMA,
          pltpu.SemaphoreType.DMA,
          pltpu.SemaphoreType.DMA,
          pltpu.SemaphoreType.DMA,
      ],
  )(ht, edges2d)


def _tc1a_body(x_ref, w1_ref, h_ref):
  h_ref[...] = jnp.dot(x_ref[...], w1_ref[...],
                       preferred_element_type=jnp.float32)


@jax.jit
def _tc1a_call(x, W1):
  return pl.pallas_call(
      _tc1a_body,
      out_shape=jax.ShapeDtypeStruct((N, FEAT), jnp.float32),
  )(x, W1)


def _tc1b_body(degp_ref, h_ref, dinv_ref, ht_ref):
  dv = degp_ref[...]
  deg = (dv[0] + dv[1] + 1.0)[:N]
  dinv = lax.rsqrt(deg)[:, None]
  dinv_ref[...] = dinv
  ht_ref[...] = h_ref[...] * dinv


@jax.jit
def _tc1b_call(degp, h1):
  return pl.pallas_call(
      _tc1b_body,
      out_shape=[
          jax.ShapeDtypeStruct((N, 1), jnp.float32),
          jax.ShapeDtypeStruct((N, FEAT), jnp.float32),
      ],
  )(degp, h1)


def _bn_relu(conv, gamma, beta):
  mu = jnp.mean(conv, axis=0, keepdims=True)
  xc = conv - mu
  var = jnp.mean(xc * xc, axis=0, keepdims=True)
  return jnp.maximum(xc * lax.rsqrt(var + 1e-5) * gamma + beta, 0.0)


def _tc2_body(agg_ref, ht_ref, dinv_ref, g_ref, b_ref, w_ref, out_ref):
  dinv = dinv_ref[...]
  conv = (agg_ref[0, :N] + agg_ref[1, :N] + ht_ref[...]) * dinv
  y = _bn_relu(conv, g_ref[...], b_ref[...])
  out_ref[...] = jnp.dot(y, w_ref[...],
                         preferred_element_type=jnp.float32) * dinv


@jax.jit
def _tc2_call(agg, ht, dinv, gamma, beta, Wn):
  return pl.pallas_call(
      _tc2_body,
      out_shape=jax.ShapeDtypeStruct((N, FEAT), jnp.float32),
  )(agg, ht, dinv, gamma, beta, Wn)


def _tc3_body(agg_ref, ht_ref, dinv_ref, g_ref, b_ref, batch_ref,
              f1w_ref, f1b_ref, f2w_ref, f2b_ref, out_ref):
  conv = (agg_ref[0, :N] + agg_ref[1, :N] + ht_ref[...]) * dinv_ref[...]
  y = _bn_relu(conv, g_ref[...], b_ref[...])
  gid = lax.broadcasted_iota(jnp.int32, (1, G), 1)
  m = (batch_ref[...] == gid).astype(jnp.float32)
  dn = (((0,), (0,)), ((), ()))
  sums = lax.dot_general(m, y, dn, preferred_element_type=jnp.float32)
  ones = jnp.ones((N, 1), jnp.float32)
  counts = lax.dot_general(m, ones, dn, preferred_element_type=jnp.float32)
  pooled = sums / jnp.maximum(counts, 1.0)
  a = jnp.maximum(
      jnp.dot(pooled, f1w_ref[...], preferred_element_type=jnp.float32)
      + f1b_ref[...], 0.0)
  res = jnp.dot(a, f2w_ref[...], preferred_element_type=jnp.float32)
  out_ref[...] = res[:, :2] + f2b_ref[...]


@jax.jit
def _tc3_call(agg, ht, dinv, gamma, beta, batch2d,
              fc1Wp, fc1bp, fc2Wp, fc2bp):
  return pl.pallas_call(
      _tc3_body,
      out_shape=jax.ShapeDtypeStruct((G, 2), jnp.float32),
  )(agg, ht, dinv, gamma, beta, batch2d, fc1Wp, fc1bp, fc2Wp, fc2bp)


def kernel(x, edge_index, batch, W1, b1, gamma1, beta1, W2, b2, gamma2,
           beta2, fc1_W, fc1_b, fc2_W, fc2_b):
  edges2d = jnp.concatenate(
      [edge_index, jnp.asarray(_PAD2)], axis=1).reshape(2, NW * CPW, CHUNK)

  degp = _deg_call(edges2d)
  h1 = _tc1a_call(x, W1)
  dinv, ht1 = _tc1b_call(degp, h1)

  agg1 = _conv_call(ht1, edges2d)
  ht2 = _tc2_call(agg1, ht1, dinv, gamma1[None, :], beta1[None, :], W2)

  agg2 = _conv_call(ht2, edges2d)

  fc1Wp = jnp.pad(fc1_W, ((0, 0), (0, FEAT - fc1_W.shape[1])))
  fc1bp = jnp.pad(fc1_b, (0, FEAT - fc1_b.shape[0]))[None, :]
  fc2Wp = jnp.pad(fc2_W, ((0, FEAT - fc2_W.shape[0]),
                          (0, FEAT - fc2_W.shape[1])))
  return _tc3_call(agg2, ht2, dinv,
                   gamma2[None, :], beta2[None, :], batch[:, None],
                   fc1Wp, fc1bp, fc2Wp, fc2_b[None, :])


# speedup vs baseline: 1.1477x; 1.0129x over previous
import functools

import jax
import jax.numpy as jnp
import numpy as np
from jax import lax
from jax.experimental import pallas as pl
from jax.experimental.pallas import tpu as pltpu
from jax.experimental.pallas import tpu_sc as plsc

N = 10000
E = 320000
FEAT = 128
G = 128

NC = 2
NS = 16
NW = NC * NS
CHUNK = 64
CPW = 160
IBLK = 32
EPAD = NW * CPW * CHUNK
ACC_ROWS = 10240
RPS = ACC_ROWS // NS
PAD_DST = N
NBLK = CPW // IBLK

_AR = np.arange(EPAD - E, dtype=np.int32)
_PAD2 = np.stack([_AR % N, PAD_DST + (_AR % (ACC_ROWS - N))])


def _deg_body(edges_hbm, out_hbm, idx_v, ones_v, zbuf_v, acc_sh, dsem):
  c = lax.axis_index("c")
  s = lax.axis_index("s")
  w = c * NS + s

  pltpu.sync_copy(edges_hbm.at[1, pl.ds(w * CPW, CPW)], idx_v)

  def _fill(i, _):
    ones_v[pl.ds(i * 16, 16)] = jnp.ones((16,), jnp.float32)
    return 0

  lax.fori_loop(0, CHUNK // 16, _fill, 0)

  def _zero(i, _):
    zbuf_v[pl.ds(i * 16, 16)] = jnp.zeros((16,), jnp.float32)
    return 0

  lax.fori_loop(0, RPS // 16, _zero, 0)

  pltpu.sync_copy(zbuf_v, acc_sh.at[pl.ds(s * RPS, RPS)])
  plsc.subcore_barrier()

  def _grp(g, _):
    for j in range(16):
      pltpu.async_copy(ones_v, acc_sh.at[idx_v.at[g * 16 + j]], dsem,
                       add=True)
    for j in range(16):
      pltpu.make_async_copy(ones_v, acc_sh.at[idx_v.at[g * 16 + j]],
                            dsem).wait()
    return 0

  lax.fori_loop(0, CPW // 16, _grp, 0)
  plsc.subcore_barrier()

  pltpu.sync_copy(acc_sh.at[pl.ds(s * RPS, RPS)],
                  out_hbm.at[c, pl.ds(s * RPS, RPS)])


@jax.jit
def _deg_call(edges2d):
  return pl.kernel(
      _deg_body,
      out_type=jax.ShapeDtypeStruct((NC, ACC_ROWS), jnp.float32),
      mesh=plsc.VectorSubcoreMesh(core_axis_name="c", subcore_axis_name="s"),
      scratch_types=[
          pltpu.VMEM((CPW, CHUNK), jnp.int32),
          pltpu.VMEM((CHUNK,), jnp.float32),
          pltpu.VMEM((RPS,), jnp.float32),
          pltpu.VMEM_SHARED((ACC_ROWS,), jnp.float32),
          pltpu.SemaphoreType.DMA,
      ],
  )(edges2d)


def _conv_body(ht_hbm, edges_hbm, out_hbm,
               sidxA_v, didxA_v, sidxB_v, didxB_v,
               rows0_v, rows1_v, rows2_v, acc_sh,
               gsem0, gsem1, gsem2, isemA, isemB):
  c = lax.axis_index("c")
  s = lax.axis_index("s")
  w = c * NS + s

  pltpu.async_copy(edges_hbm.at[0, pl.ds(w * CPW, IBLK)], sidxA_v, isemA)
  pltpu.async_copy(edges_hbm.at[1, pl.ds(w * CPW, IBLK)], didxA_v, isemA)

  def _zero(i, _):
    r = i // 8
    q = i % 8
    rows0_v[r, pl.ds(q * 16, 16)] = jnp.zeros((16,), jnp.float32)
    return 0

  lax.fori_loop(0, CHUNK * 8, _zero, 0)
  for k in range(RPS // CHUNK):
    pltpu.async_copy(rows0_v, acc_sh.at[pl.ds(s * RPS + k * CHUNK, CHUNK)],
                     gsem0)
  for k in range(RPS // CHUNK):
    pltpu.make_async_copy(
        rows0_v, acc_sh.at[pl.ds(s * RPS + k * CHUNK, CHUNK)], gsem0).wait()
  plsc.subcore_barrier()

  bufs = (rows0_v, rows1_v, rows2_v)
  sems = (gsem0, gsem1, gsem2)
  idxs = ((sidxA_v, didxA_v, isemA), (sidxB_v, didxB_v, isemB))

  def _stage(b, sync):
    si, di, sem = idxs[b % 2]
    base = w * CPW + b * IBLK
    if sync:
      pltpu.sync_copy(edges_hbm.at[0, pl.ds(base, IBLK)], si)
      pltpu.sync_copy(edges_hbm.at[1, pl.ds(base, IBLK)], di)
    else:
      pltpu.async_copy(edges_hbm.at[0, pl.ds(base, IBLK)], si, sem)
      pltpu.async_copy(edges_hbm.at[1, pl.ds(base, IBLK)], di, sem)

  def _stage_wait(b):
    si, di, sem = idxs[b % 2]
    base = w * CPW + b * IBLK
    pltpu.make_async_copy(edges_hbm.at[0, pl.ds(base, IBLK)], si, sem).wait()
    pltpu.make_async_copy(edges_hbm.at[1, pl.ds(base, IBLK)], di, sem).wait()

  def _gather(g):
    si = idxs[(g // IBLK) % 2][0]
    k = g % 3
    pltpu.async_copy(ht_hbm.at[si.at[g % IBLK]], bufs[k], sems[k])

  def _gwait(g):
    si = idxs[(g // IBLK) % 2][0]
    k = g % 3
    pltpu.make_async_copy(ht_hbm.at[si.at[g % IBLK]], bufs[k],
                          sems[k]).wait()

  if NBLK > 1:
    _stage(1, sync=False)
  _stage_wait(0)
  _gather(0)
  _gather(1)
  for jj in range(CPW):
    g2 = jj + 2
    if g2 < CPW:
      if g2 % IBLK == 0:
        _stage_wait(g2 // IBLK)
      _gather(g2)
    _gwait(jj)
    di = idxs[(jj // IBLK) % 2][1]
    pltpu.sync_copy(bufs[jj % 3], acc_sh.at[di.at[jj % IBLK]], add=True)
    if jj % IBLK == IBLK - 1 and jj // IBLK + 2 < NBLK:
      _stage(jj // IBLK + 2, sync=False)
  plsc.subcore_barrier()

  pltpu.sync_copy(acc_sh.at[pl.ds(s * RPS, RPS)],
                  out_hbm.at[c, pl.ds(s * RPS, RPS)])


@jax.jit
def _conv_call(ht, edges2d):
  return pl.kernel(
      _conv_body,
      out_type=jax.ShapeDtypeStruct((NC, ACC_ROWS, FEAT), jnp.float32),
      mesh=plsc.VectorSubcoreMesh(core_axis_name="c", subcore_axis_name="s"),
      scratch_types=[
          pltpu.VMEM((IBLK, CHUNK), jnp.int32),
          pltpu.VMEM((IBLK, CHUNK), jnp.int32),
          pltpu.VMEM((IBLK, CHUNK), jnp.int32),
          pltpu.VMEM((IBLK, CHUNK), jnp.int32),
          pltpu.VMEM((CHUNK, FEAT), jnp.float32),
          pltpu.VMEM((CHUNK, FEAT), jnp.float32),
          pltpu.VMEM((CHUNK, FEAT), jnp.float32),
          pltpu.VMEM_SHARED((ACC_ROWS, FEAT), jnp.float32),
          pltpu.SemaphoreType.DMA,
          pltpu.SemaphoreType.DMA,
          pltpu.SemaphoreType.DMA,
          pltpu.SemaphoreType.DMA,
          pltpu.SemaphoreType.DMA,
      ],
  )(ht, edges2d)


def _tc1a_body(x_ref, w1_ref, h_ref):
  h_ref[...] = jnp.dot(x_ref[...], w1_ref[...],
                       preferred_element_type=jnp.float32)


@jax.jit
def _tc1a_call(x, W1):
  return pl.pallas_call(
      _tc1a_body,
      out_shape=jax.ShapeDtypeStruct((N, FEAT), jnp.float32),
  )(x, W1)


def _tc1b_body(degp_ref, h_ref, dinv_ref, ht_ref):
  dv = degp_ref[...]
  deg = (dv[0] + dv[1] + 1.0)[:N]
  dinv = lax.rsqrt(deg)[:, None]
  dinv_ref[...] = dinv
  ht_ref[...] = h_ref[...] * dinv


@jax.jit
def _tc1b_call(degp, h1):
  return pl.pallas_call(
      _tc1b_body,
      out_shape=[
          jax.ShapeDtypeStruct((N, 1), jnp.float32),
          jax.ShapeDtypeStruct((N, FEAT), jnp.float32),
      ],
  )(degp, h1)


def _bn_relu(conv, gamma, beta):
  mu = jnp.mean(conv, axis=0, keepdims=True)
  xc = conv - mu
  var = jnp.mean(xc * xc, axis=0, keepdims=True)
  return jnp.maximum(xc * lax.rsqrt(var + 1e-5) * gamma + beta, 0.0)


def _tc2_body(agg_ref, ht_ref, dinv_ref, g_ref, b_ref, w_ref, out_ref):
  dinv = dinv_ref[...]
  conv = (agg_ref[0, :N] + agg_ref[1, :N] + ht_ref[...]) * dinv
  y = _bn_relu(conv, g_ref[...], b_ref[...])
  out_ref[...] = jnp.dot(y, w_ref[...],
                         preferred_element_type=jnp.float32) * dinv


@jax.jit
def _tc2_call(agg, ht, dinv, gamma, beta, Wn):
  return pl.pallas_call(
      _tc2_body,
      out_shape=jax.ShapeDtypeStruct((N, FEAT), jnp.float32),
  )(agg, ht, dinv, gamma, beta, Wn)


def _tc3_body(agg_ref, ht_ref, dinv_ref, g_ref, b_ref, batch_ref,
              f1w_ref, f1b_ref, f2w_ref, f2b_ref, out_ref):
  conv = (agg_ref[0, :N] + agg_ref[1, :N] + ht_ref[...]) * dinv_ref[...]
  y = _bn_relu(conv, g_ref[...], b_ref[...])
  gid = lax.broadcasted_iota(jnp.int32, (1, G), 1)
  m = (batch_ref[...] == gid).astype(jnp.float32)
  dn = (((0,), (0,)), ((), ()))
  sums = lax.dot_general(m, y, dn, preferred_element_type=jnp.float32)
  ones = jnp.ones((N, 1), jnp.float32)
  counts = lax.dot_general(m, ones, dn, preferred_element_type=jnp.float32)
  pooled = sums / jnp.maximum(counts, 1.0)
  a = jnp.maximum(
      jnp.dot(pooled, f1w_ref[...], preferred_element_type=jnp.float32)
      + f1b_ref[...], 0.0)
  res = jnp.dot(a, f2w_ref[...], preferred_element_type=jnp.float32)
  out_ref[...] = res[:, :2] + f2b_ref[...]


@jax.jit
def _tc3_call(agg, ht, dinv, gamma, beta, batch2d,
              fc1Wp, fc1bp, fc2Wp, fc2bp):
  return pl.pallas_call(
      _tc3_body,
      out_shape=jax.ShapeDtypeStruct((G, 2), jnp.float32),
  )(agg, ht, dinv, gamma, beta, batch2d, fc1Wp, fc1bp, fc2Wp, fc2bp)


def kernel(x, edge_index, batch, W1, b1, gamma1, beta1, W2, b2, gamma2,
           beta2, fc1_W, fc1_b, fc2_W, fc2_b):
  edges2d = jnp.concatenate(
      [edge_index, jnp.asarray(_PAD2)], axis=1).reshape(2, NW * CPW, CHUNK)

  degp = _deg_call(edges2d)
  h1 = _tc1a_call(x, W1)
  dinv, ht1 = _tc1b_call(degp, h1)

  agg1 = _conv_call(ht1, edges2d)
  ht2 = _tc2_call(agg1, ht1, dinv, gamma1[None, :], beta1[None, :], W2)

  agg2 = _conv_call(ht2, edges2d)

  fc1Wp = jnp.pad(fc1_W, ((0, 0), (0, FEAT - fc1_W.shape[1])))
  fc1bp = jnp.pad(fc1_b, (0, FEAT - fc1_b.shape[0]))[None, :]
  fc2Wp = jnp.pad(fc2_W, ((0, FEAT - fc2_W.shape[0]),
                          (0, FEAT - fc2_W.shape[1])))
  return _tc3_call(agg2, ht2, dinv,
                   gamma2[None, :], beta2[None, :], batch[:, None],
                   fc1Wp, fc1bp, fc2Wp, fc2_b[None, :])


# speedup vs baseline: 1.1494x; 1.0015x over previous
import jax
import jax.numpy as jnp
import numpy as np
from jax import lax
from jax.experimental import pallas as pl
from jax.experimental.pallas import tpu as pltpu
from jax.experimental.pallas import tpu_sc as plsc

N = 10000
E = 320000
FEAT = 128
G = 128

NC = 2
NS = 16
NW = NC * NS
CHUNK = 64
CPW = 160
IBLK = 32
EPAD = NW * CPW * CHUNK
ACC_ROWS = 10240
RPS = ACC_ROWS // NS
PAD_DST = N
NBLK = CPW // IBLK

_AR = np.arange(EPAD - E, dtype=np.int32)
_PAD2 = np.stack([_AR % N, PAD_DST + (_AR % (ACC_ROWS - N))])


def _deg_body(edges_hbm, out_hbm, idx_v, ones_v, zbuf_v, acc_sh, dsem):
  c = lax.axis_index("c")
  s = lax.axis_index("s")
  w = c * NS + s

  pltpu.sync_copy(edges_hbm.at[1, pl.ds(w * CPW, CPW)], idx_v)

  def _fill(i, _):
    ones_v[pl.ds(i * 16, 16)] = jnp.ones((16,), jnp.float32)
    return 0

  lax.fori_loop(0, CHUNK // 16, _fill, 0)

  def _zero(i, _):
    zbuf_v[pl.ds(i * 16, 16)] = jnp.zeros((16,), jnp.float32)
    return 0

  lax.fori_loop(0, RPS // 16, _zero, 0)

  pltpu.sync_copy(zbuf_v, acc_sh.at[pl.ds(s * RPS, RPS)])
  plsc.subcore_barrier()

  def _grp(g, _):
    for j in range(16):
      pltpu.async_copy(ones_v, acc_sh.at[idx_v.at[g * 16 + j]], dsem,
                       add=True)
    for j in range(16):
      pltpu.make_async_copy(ones_v, acc_sh.at[idx_v.at[g * 16 + j]],
                            dsem).wait()
    return 0

  lax.fori_loop(0, CPW // 16, _grp, 0)
  plsc.subcore_barrier()

  pltpu.sync_copy(acc_sh.at[pl.ds(s * RPS, RPS)],
                  out_hbm.at[c, pl.ds(s * RPS, RPS)])


@jax.jit
def _deg_call(edges2d):
  return pl.kernel(
      _deg_body,
      out_type=jax.ShapeDtypeStruct((NC, ACC_ROWS), jnp.float32),
      mesh=plsc.VectorSubcoreMesh(core_axis_name="c", subcore_axis_name="s"),
      scratch_types=[
          pltpu.VMEM((CPW, CHUNK), jnp.int32),
          pltpu.VMEM((CHUNK,), jnp.float32),
          pltpu.VMEM((RPS,), jnp.float32),
          pltpu.VMEM_SHARED((ACC_ROWS,), jnp.float32),
          pltpu.SemaphoreType.DMA,
      ],
  )(edges2d)


def _conv_body(ht_hbm, edges_hbm, out_hbm,
               sidxA_v, didxA_v, sidxB_v, didxB_v,
               rows0_v, rows1_v, rows2_v, acc_sh,
               gsem0, gsem1, gsem2, isemA, isemB):
  c = lax.axis_index("c")
  s = lax.axis_index("s")
  w = c * NS + s

  pltpu.async_copy(edges_hbm.at[0, pl.ds(w * CPW, IBLK)], sidxA_v, isemA)
  pltpu.async_copy(edges_hbm.at[1, pl.ds(w * CPW, IBLK)], didxA_v, isemA)

  def _zero(i, _):
    r = i // 8
    q = i % 8
    rows0_v[r, pl.ds(q * 16, 16)] = jnp.zeros((16,), jnp.float32)
    return 0

  lax.fori_loop(0, CHUNK * 8, _zero, 0)
  for k in range(RPS // CHUNK):
    pltpu.async_copy(rows0_v, acc_sh.at[pl.ds(s * RPS + k * CHUNK, CHUNK)],
                     gsem0)
  for k in range(RPS // CHUNK):
    pltpu.make_async_copy(
        rows0_v, acc_sh.at[pl.ds(s * RPS + k * CHUNK, CHUNK)], gsem0).wait()
  plsc.subcore_barrier()

  bufs = (rows0_v, rows1_v, rows2_v)
  sems = (gsem0, gsem1, gsem2)
  idxs = ((sidxA_v, didxA_v, isemA), (sidxB_v, didxB_v, isemB))

  def _stage(b, sync):
    si, di, sem = idxs[b % 2]
    base = w * CPW + b * IBLK
    if sync:
      pltpu.sync_copy(edges_hbm.at[0, pl.ds(base, IBLK)], si)
      pltpu.sync_copy(edges_hbm.at[1, pl.ds(base, IBLK)], di)
    else:
      pltpu.async_copy(edges_hbm.at[0, pl.ds(base, IBLK)], si, sem)
      pltpu.async_copy(edges_hbm.at[1, pl.ds(base, IBLK)], di, sem)

  def _stage_wait(b):
    si, di, sem = idxs[b % 2]
    base = w * CPW + b * IBLK
    pltpu.make_async_copy(edges_hbm.at[0, pl.ds(base, IBLK)], si, sem).wait()
    pltpu.make_async_copy(edges_hbm.at[1, pl.ds(base, IBLK)], di, sem).wait()

  def _gather(g):
    si = idxs[(g // IBLK) % 2][0]
    k = g % 3
    pltpu.async_copy(ht_hbm.at[si.at[g % IBLK]], bufs[k], sems[k])

  def _gwait(g):
    si = idxs[(g // IBLK) % 2][0]
    k = g % 3
    pltpu.make_async_copy(ht_hbm.at[si.at[g % IBLK]], bufs[k],
                          sems[k]).wait()

  if NBLK > 1:
    _stage(1, sync=False)
  _stage_wait(0)
  _gather(0)
  _gather(1)
  for jj in range(CPW):
    g2 = jj + 2
    if g2 < CPW:
      if g2 % IBLK == 0:
        _stage_wait(g2 // IBLK)
      _gather(g2)
    _gwait(jj)
    di = idxs[(jj // IBLK) % 2][1]
    pltpu.sync_copy(bufs[jj % 3], acc_sh.at[di.at[jj % IBLK]], add=True)
    if jj % IBLK == IBLK - 1 and jj // IBLK + 2 < NBLK:
      _stage(jj // IBLK + 2, sync=False)
  plsc.subcore_barrier()

  pltpu.sync_copy(acc_sh.at[pl.ds(s * RPS, RPS)],
                  out_hbm.at[c, pl.ds(s * RPS, RPS)])


@jax.jit
def _conv_call(ht, edges2d):
  return pl.kernel(
      _conv_body,
      out_type=jax.ShapeDtypeStruct((NC, ACC_ROWS, FEAT), jnp.float32),
      mesh=plsc.VectorSubcoreMesh(core_axis_name="c", subcore_axis_name="s"),
      scratch_types=[
          pltpu.VMEM((IBLK, CHUNK), jnp.int32),
          pltpu.VMEM((IBLK, CHUNK), jnp.int32),
          pltpu.VMEM((IBLK, CHUNK), jnp.int32),
          pltpu.VMEM((IBLK, CHUNK), jnp.int32),
          pltpu.VMEM((CHUNK, FEAT), jnp.float32),
          pltpu.VMEM((CHUNK, FEAT), jnp.float32),
          pltpu.VMEM((CHUNK, FEAT), jnp.float32),
          pltpu.VMEM_SHARED((ACC_ROWS, FEAT), jnp.float32),
          pltpu.SemaphoreType.DMA,
          pltpu.SemaphoreType.DMA,
          pltpu.SemaphoreType.DMA,
          pltpu.SemaphoreType.DMA,
          pltpu.SemaphoreType.DMA,
      ],
  )(ht, edges2d)


def _tc1a_body(x_ref, w1_ref, h_ref):
  h_ref[...] = jnp.dot(x_ref[...], w1_ref[...],
                       preferred_element_type=jnp.float32)


@jax.jit
def _tc1a_call(x, W1):
  return pl.pallas_call(
      _tc1a_body,
      out_shape=jax.ShapeDtypeStruct((N, FEAT), jnp.float32),
  )(x, W1)


def _tc1b_body(degp_ref, h_ref, dinv_ref, ht_ref):
  dv = degp_ref[...]
  deg = (dv[0] + dv[1] + 1.0)[:N]
  dinv = lax.rsqrt(deg)[:, None]
  dinv_ref[...] = dinv
  ht_ref[...] = h_ref[...] * dinv


@jax.jit
def _tc1b_call(degp, h1):
  return pl.pallas_call(
      _tc1b_body,
      out_shape=[
          jax.ShapeDtypeStruct((N, 1), jnp.float32),
          jax.ShapeDtypeStruct((N, FEAT), jnp.float32),
      ],
  )(degp, h1)


def _bn_relu(conv, gamma, beta):
  mu = jnp.mean(conv, axis=0, keepdims=True)
  xc = conv - mu
  var = jnp.mean(xc * xc, axis=0, keepdims=True)
  return jnp.maximum(xc * lax.rsqrt(var + 1e-5) * gamma + beta, 0.0)


def _tc2_body(agg_ref, ht_ref, dinv_ref, g_ref, b_ref, w_ref, out_ref):
  dinv = dinv_ref[...]
  conv = (agg_ref[0, :N] + agg_ref[1, :N] + ht_ref[...]) * dinv
  y = _bn_relu(conv, g_ref[...], b_ref[...])
  out_ref[...] = jnp.dot(y, w_ref[...],
                         preferred_element_type=jnp.float32) * dinv


@jax.jit
def _tc2_call(agg, ht, dinv, gamma, beta, Wn):
  return pl.pallas_call(
      _tc2_body,
      out_shape=jax.ShapeDtypeStruct((N, FEAT), jnp.float32),
  )(agg, ht, dinv, gamma, beta, Wn)


def _tc3_body(agg_ref, ht_ref, dinv_ref, g_ref, b_ref, batch_ref,
              f1w_ref, f1b_ref, f2w_ref, f2b_ref, out_ref):
  conv = (agg_ref[0, :N] + agg_ref[1, :N] + ht_ref[...]) * dinv_ref[...]
  y = _bn_relu(conv, g_ref[...], b_ref[...])
  gid = lax.broadcasted_iota(jnp.int32, (1, G), 1)
  m = (batch_ref[...] == gid).astype(jnp.float32)
  dn = (((0,), (0,)), ((), ()))
  sums = lax.dot_general(m, y, dn, preferred_element_type=jnp.float32)
  ones = jnp.ones((N, 1), jnp.float32)
  counts = lax.dot_general(m, ones, dn, preferred_element_type=jnp.float32)
  pooled = sums / jnp.maximum(counts, 1.0)
  a = jnp.maximum(
      jnp.dot(pooled, f1w_ref[...], preferred_element_type=jnp.float32)
      + f1b_ref[...], 0.0)
  res = jnp.dot(a, f2w_ref[...], preferred_element_type=jnp.float32)
  out_ref[...] = res[:, :2] + f2b_ref[...]


@jax.jit
def _tc3_call(agg, ht, dinv, gamma, beta, batch2d,
              fc1Wp, fc1bp, fc2Wp, fc2bp):
  return pl.pallas_call(
      _tc3_body,
      out_shape=jax.ShapeDtypeStruct((G, 2), jnp.float32),
  )(agg, ht, dinv, gamma, beta, batch2d, fc1Wp, fc1bp, fc2Wp, fc2bp)


def kernel(x, edge_index, batch, W1, b1, gamma1, beta1, W2, b2, gamma2,
           beta2, fc1_W, fc1_b, fc2_W, fc2_b):
  edges2d = jnp.concatenate(
      [edge_index, jnp.asarray(_PAD2)], axis=1).reshape(2, NW * CPW, CHUNK)

  degp = _deg_call(edges2d)
  h1 = _tc1a_call(x, W1)
  dinv, ht1 = _tc1b_call(degp, h1)

  agg1 = _conv_call(ht1, edges2d)
  ht2 = _tc2_call(agg1, ht1, dinv, gamma1[None, :], beta1[None, :], W2)

  agg2 = _conv_call(ht2, edges2d)

  fc1Wp = jnp.pad(fc1_W, ((0, 0), (0, FEAT - fc1_W.shape[1])))
  fc1bp = jnp.pad(fc1_b, (0, FEAT - fc1_b.shape[0]))[None, :]
  fc2Wp = jnp.pad(fc2_W, ((0, FEAT - fc2_W.shape[0]),
                          (0, FEAT - fc2_W.shape[1])))
  return _tc3_call(agg2, ht2, dinv,
                   gamma2[None, :], beta2[None, :], batch[:, None],
                   fc1Wp, fc1bp, fc2Wp, fc2_b[None, :])
